# baseline (speedup 1.0000x reference)
import functools

import jax
import jax.numpy as jnp
from jax import lax
from jax.experimental import pallas as pl
from jax.experimental.pallas import tpu as pltpu
from jax.experimental.pallas import tpu_sc as plsc

N_ATOMS = 320000
D_FEAT = 128
N_MOLS = 10000

N_CORES = 2
N_SUB = 16
NW = N_CORES * N_SUB
PER_W = N_ATOMS // NW
CHUNK = 125
STEPS = PER_W // CHUNK
M_PER_SUB = N_MOLS // N_SUB
ZROWS = 125
ZCHUNKS = N_MOLS // ZROWS


def _sc_partials(features4, idx3, zeros_stage):
    mesh = plsc.VectorSubcoreMesh(core_axis_name="c", subcore_axis_name="s")

    @functools.partial(
        pl.kernel,
        out_type=jax.ShapeDtypeStruct((N_CORES, ZCHUNKS, ZROWS, D_FEAT),
                                      jnp.float32),
        mesh=mesh,
        scratch_types=[
            pltpu.VMEM((STEPS, CHUNK), jnp.int32),
            pltpu.VMEM((CHUNK, D_FEAT), jnp.float32),
            pltpu.VMEM((CHUNK, D_FEAT), jnp.float32),
            pltpu.VMEM_SHARED((N_MOLS, D_FEAT), jnp.float32),
            pltpu.SemaphoreType.DMA,
            pltpu.SemaphoreType.DMA,
        ],
    )
    def k(feat_hbm, idx_hbm, zero_hbm, part_hbm,
          idx_v, rows_a, rows_b, accum_sh, sem_a, sem_b):
        c = lax.axis_index("c")
        s = lax.axis_index("s")
        wid = c * N_SUB + s

        pltpu.sync_copy(zero_hbm, rows_a)
        for kk in range(M_PER_SUB // ZROWS):
            pltpu.sync_copy(rows_a,
                            accum_sh.at[pl.ds(s * M_PER_SUB + kk * ZROWS, ZROWS)])
        plsc.subcore_barrier()

        pltpu.sync_copy(idx_hbm.at[wid], idx_v)

        def step(j, carry):
            pltpu.sync_copy(feat_hbm.at[wid, j], rows_a)
            pltpu.sync_copy(rows_a, accum_sh.at[idx_v.at[j]], add=True)
            return carry

        lax.fori_loop(0, STEPS, step, 0)
        plsc.subcore_barrier()

        for kk in range(M_PER_SUB // ZROWS):
            q = s * (M_PER_SUB // ZROWS) + kk
            pltpu.sync_copy(accum_sh.at[pl.ds(q * ZROWS, ZROWS)], rows_a)
            pltpu.sync_copy(rows_a, part_hbm.at[c, q])

    return k(features4, idx3, zeros_stage)


def _combine_body(a_ref, b_ref, o_ref):
    o_ref[...] = a_ref[...] + b_ref[...]


_COMBINE_BLK = 1000


def _combine(p0, p1):
    grid = N_MOLS // _COMBINE_BLK
    spec = pl.BlockSpec((_COMBINE_BLK, D_FEAT), lambda i: (i, 0))
    return pl.pallas_call(
        _combine_body,
        grid=(grid,),
        in_specs=[spec, spec],
        out_specs=spec,
        out_shape=jax.ShapeDtypeStruct((N_MOLS, D_FEAT), jnp.float32),
    )(p0, p1)


def kernel(features, mol_index, n_molecules):
    del n_molecules
    feat4 = features.reshape(NW, STEPS, CHUNK, D_FEAT)
    idx3 = mol_index.astype(jnp.int32).reshape(NW, STEPS, CHUNK)
    zeros_stage = jnp.zeros((ZROWS, D_FEAT), jnp.float32)
    part = _sc_partials(feat4, idx3, zeros_stage)
    part = part.reshape(N_CORES, N_MOLS, D_FEAT)
    return _combine(part[0], part[1])


# speedup vs baseline: 1.8111x; 1.8111x over previous
import functools

import jax
import jax.numpy as jnp
from jax import lax
from jax.experimental import pallas as pl
from jax.experimental.pallas import tpu as pltpu
from jax.experimental.pallas import tpu_sc as plsc

N_ATOMS = 320000
D_FEAT = 128
N_MOLS = 10000

N_CORES = 2
N_SUB = 16
NW = N_CORES * N_SUB
PER_W = N_ATOMS // NW
CHUNK = 80
STEPS = PER_W // CHUNK
ZROWS = 80
ZCHUNKS = N_MOLS // ZROWS
ZITERS = -(-ZCHUNKS // N_SUB)


def _sc_partials(features4, idx3, zeros_stage):
    mesh = plsc.VectorSubcoreMesh(core_axis_name="c", subcore_axis_name="s")

    @functools.partial(
        pl.kernel,
        out_type=jax.ShapeDtypeStruct((N_CORES, ZCHUNKS, ZROWS, D_FEAT),
                                      jnp.float32),
        mesh=mesh,
        scratch_types=[
            pltpu.VMEM((STEPS, CHUNK), jnp.int32),
            pltpu.VMEM((CHUNK, D_FEAT), jnp.float32),
            pltpu.VMEM((CHUNK, D_FEAT), jnp.float32),
            pltpu.VMEM_SHARED((N_MOLS, D_FEAT), jnp.float32),
            pltpu.SemaphoreType.DMA,
            pltpu.SemaphoreType.DMA,
        ],
    )
    def k(feat_hbm, idx_hbm, zero_hbm, part_hbm,
          idx_v, rows_a, rows_b, accum_sh, sem_a, sem_b):
        c = lax.axis_index("c")
        s = lax.axis_index("s")
        wid = c * N_SUB + s

        pltpu.sync_copy(zero_hbm, rows_a)
        for kk in range(ZITERS):
            q = kk * N_SUB + s

            @pl.when(q < ZCHUNKS)
            def _():
                pltpu.sync_copy(rows_a, accum_sh.at[pl.ds(q * ZROWS, ZROWS)])

        plsc.subcore_barrier()

        pltpu.sync_copy(idx_hbm.at[wid], idx_v)

        def scatter_wait(buf, sem):
            pltpu.make_async_copy(buf, accum_sh.at[idx_v.at[0]], sem).wait()

        def step(t, carry):
            j = 2 * t

            @pl.when(t > 0)
            def _():
                scatter_wait(rows_a, sem_a)

            pltpu.sync_copy(feat_hbm.at[wid, j], rows_a)

            @pl.when(t > 0)
            def _():
                scatter_wait(rows_b, sem_b)

            pltpu.async_copy(rows_a, accum_sh.at[idx_v.at[j]], sem_a, add=True)
            pltpu.sync_copy(feat_hbm.at[wid, j + 1], rows_b)
            pltpu.async_copy(rows_b, accum_sh.at[idx_v.at[j + 1]], sem_b,
                             add=True)
            return carry

        lax.fori_loop(0, (STEPS - 1) // 2, step, 0)
        scatter_wait(rows_a, sem_a)
        scatter_wait(rows_b, sem_b)
        pltpu.sync_copy(feat_hbm.at[wid, STEPS - 1], rows_a)
        pltpu.sync_copy(rows_a, accum_sh.at[idx_v.at[STEPS - 1]], add=True)
        plsc.subcore_barrier()

        for kk in range(ZITERS):
            q = kk * N_SUB + s

            @pl.when(q < ZCHUNKS)
            def _():
                pltpu.sync_copy(accum_sh.at[pl.ds(q * ZROWS, ZROWS)], rows_a)
                pltpu.sync_copy(rows_a, part_hbm.at[c, q])

    return k(features4, idx3, zeros_stage)


def _combine_body(a_ref, b_ref, o_ref):
    o_ref[...] = a_ref[...] + b_ref[...]


_COMBINE_BLK = 1000


def _combine(p0, p1):
    grid = N_MOLS // _COMBINE_BLK
    spec = pl.BlockSpec((_COMBINE_BLK, D_FEAT), lambda i: (i, 0))
    return pl.pallas_call(
        _combine_body,
        grid=(grid,),
        in_specs=[spec, spec],
        out_specs=spec,
        out_shape=jax.ShapeDtypeStruct((N_MOLS, D_FEAT), jnp.float32),
    )(p0, p1)


def kernel(features, mol_index, n_molecules):
    del n_molecules
    feat4 = features.reshape(NW, STEPS, CHUNK, D_FEAT)
    idx3 = mol_index.astype(jnp.int32).reshape(NW, STEPS, CHUNK)
    zeros_stage = jnp.zeros((ZROWS, D_FEAT), jnp.float32)
    part = _sc_partials(feat4, idx3, zeros_stage)
    part = part.reshape(N_CORES, N_MOLS, D_FEAT)
    return _combine(part[0], part[1])


# speedup vs baseline: 1.8160x; 1.0027x over previous
import functools

import jax
import jax.numpy as jnp
from jax import lax
from jax.experimental import pallas as pl
from jax.experimental.pallas import tpu as pltpu
from jax.experimental.pallas import tpu_sc as plsc

N_ATOMS = 320000
D_FEAT = 128
N_MOLS = 10000

N_CORES = 2
N_SUB = 16
NW = N_CORES * N_SUB
PER_W = N_ATOMS // NW
CHUNK = 80
STEPS = PER_W // CHUNK
ZROWS = 80
ZCHUNKS = N_MOLS // ZROWS
ZITERS = -(-ZCHUNKS // N_SUB)


def _sc_partials(features4, idx3, zeros_stage):
    mesh = plsc.VectorSubcoreMesh(core_axis_name="c", subcore_axis_name="s")

    @functools.partial(
        pl.kernel,
        out_type=jax.ShapeDtypeStruct((N_CORES, ZCHUNKS, ZROWS, D_FEAT),
                                      jnp.float32),
        mesh=mesh,
        scratch_types=[
            pltpu.VMEM((STEPS, CHUNK), jnp.int32),
            pltpu.VMEM((CHUNK, D_FEAT), jnp.float32),
            pltpu.VMEM((CHUNK, D_FEAT), jnp.float32),
            pltpu.VMEM_SHARED((N_MOLS, D_FEAT), jnp.float32),
            pltpu.SemaphoreType.DMA,
            pltpu.SemaphoreType.DMA,
            pltpu.SemaphoreType.DMA,
            pltpu.SemaphoreType.DMA,
        ],
    )
    def k(feat_hbm, idx_hbm, zero_hbm, part_hbm,
          idx_v, rows_a, rows_b, accum_sh, sem_a, sem_b, sem_fa, sem_fb):
        c = lax.axis_index("c")
        s = lax.axis_index("s")
        wid = c * N_SUB + s

        pltpu.sync_copy(zero_hbm, rows_a)
        for kk in range(ZITERS):
            q = kk * N_SUB + s

            @pl.when(q < ZCHUNKS)
            def _():
                pltpu.sync_copy(rows_a, accum_sh.at[pl.ds(q * ZROWS, ZROWS)])

        plsc.subcore_barrier()

        pltpu.sync_copy(idx_hbm.at[wid], idx_v)

        def scatter_wait(buf, sem):
            pltpu.make_async_copy(buf, accum_sh.at[idx_v.at[0]], sem).wait()

        def fetch_wait(jj, buf, sem):
            pltpu.make_async_copy(feat_hbm.at[wid, jj], buf, sem).wait()

        pltpu.async_copy(feat_hbm.at[wid, 0], rows_a, sem_fa)

        def step(t, carry):
            j = 2 * t
            fetch_wait(j, rows_a, sem_fa)
            pltpu.async_copy(rows_a, accum_sh.at[idx_v.at[j]], sem_a, add=True)

            @pl.when(t > 0)
            def _():
                scatter_wait(rows_b, sem_b)

            pltpu.async_copy(feat_hbm.at[wid, j + 1], rows_b, sem_fb)
            fetch_wait(j + 1, rows_b, sem_fb)
            pltpu.async_copy(rows_b, accum_sh.at[idx_v.at[j + 1]], sem_b,
                             add=True)
            scatter_wait(rows_a, sem_a)

            @pl.when(j + 2 < STEPS)
            def _():
                pltpu.async_copy(feat_hbm.at[wid, j + 2], rows_a, sem_fa)

            return carry

        lax.fori_loop(0, (STEPS - 1) // 2, step, 0)
        scatter_wait(rows_b, sem_b)
        fetch_wait(STEPS - 1, rows_a, sem_fa)
        pltpu.sync_copy(rows_a, accum_sh.at[idx_v.at[STEPS - 1]], add=True)
        plsc.subcore_barrier()

        for kk in range(ZITERS):
            q = kk * N_SUB + s

            @pl.when(q < ZCHUNKS)
            def _():
                pltpu.sync_copy(accum_sh.at[pl.ds(q * ZROWS, ZROWS)], rows_a)
                pltpu.sync_copy(rows_a, part_hbm.at[c, q])

    return k(features4, idx3, zeros_stage)


def _combine_body(a_ref, b_ref, o_ref):
    o_ref[...] = a_ref[...] + b_ref[...]


_COMBINE_BLK = 1000


def _combine(p0, p1):
    grid = N_MOLS // _COMBINE_BLK
    spec = pl.BlockSpec((_COMBINE_BLK, D_FEAT), lambda i: (i, 0))
    return pl.pallas_call(
        _combine_body,
        grid=(grid,),
        in_specs=[spec, spec],
        out_specs=spec,
        out_shape=jax.ShapeDtypeStruct((N_MOLS, D_FEAT), jnp.float32),
    )(p0, p1)


def kernel(features, mol_index, n_molecules):
    del n_molecules
    feat4 = features.reshape(NW, STEPS, CHUNK, D_FEAT)
    idx3 = mol_index.astype(jnp.int32).reshape(NW, STEPS, CHUNK)
    zeros_stage = jnp.zeros((ZROWS, D_FEAT), jnp.float32)
    part = _sc_partials(feat4, idx3, zeros_stage)
    part = part.reshape(N_CORES, N_MOLS, D_FEAT)
    return _combine(part[0], part[1])


# speedup vs baseline: 2.4323x; 1.3394x over previous
import functools

import jax
import jax.numpy as jnp
from jax import lax
from jax.experimental import pallas as pl
from jax.experimental.pallas import tpu as pltpu
from jax.experimental.pallas import tpu_sc as plsc

N_ATOMS = 320000
D_FEAT = 128
N_MOLS = 10000

N_CORES = 2
N_SUB = 16
NW = N_CORES * N_SUB
PER_W = N_ATOMS // NW
CHUNK = 80
STEPS = PER_W // CHUNK
ZROWS = 80
ZCHUNKS = N_MOLS // ZROWS
ZITERS = -(-ZCHUNKS // N_SUB)


def _sc_partials(features4, idx3, zeros_stage):
    mesh = plsc.VectorSubcoreMesh(core_axis_name="c", subcore_axis_name="s")

    @functools.partial(
        pl.kernel,
        out_type=jax.ShapeDtypeStruct((N_CORES, ZCHUNKS, ZROWS, D_FEAT),
                                      jnp.float32),
        mesh=mesh,
        scratch_types=[
            pltpu.VMEM((STEPS, CHUNK), jnp.int32),
            pltpu.VMEM((CHUNK, D_FEAT), jnp.float32),
            pltpu.VMEM((CHUNK, D_FEAT), jnp.float32),
            pltpu.VMEM((CHUNK, D_FEAT), jnp.float32),
            pltpu.VMEM_SHARED((N_MOLS, D_FEAT), jnp.float32),
            pltpu.SemaphoreType.DMA,
            pltpu.SemaphoreType.DMA,
            pltpu.SemaphoreType.DMA,
            pltpu.SemaphoreType.DMA,
            pltpu.SemaphoreType.DMA,
            pltpu.SemaphoreType.DMA,
        ],
    )
    def k(feat_hbm, idx_hbm, zero_hbm, part_hbm,
          idx_v, rows_0, rows_1, rows_2, accum_sh,
          sem_s0, sem_s1, sem_s2, sem_f0, sem_f1, sem_f2):
        bufs = (rows_0, rows_1, rows_2)
        ssems = (sem_s0, sem_s1, sem_s2)
        fsems = (sem_f0, sem_f1, sem_f2)
        rows_a = rows_0
        c = lax.axis_index("c")
        s = lax.axis_index("s")
        wid = c * N_SUB + s

        pltpu.sync_copy(zero_hbm, rows_a)
        for kk in range(ZITERS):
            q = kk * N_SUB + s

            @pl.when(q < ZCHUNKS)
            def _():
                pltpu.sync_copy(rows_a, accum_sh.at[pl.ds(q * ZROWS, ZROWS)])

        plsc.subcore_barrier()

        pltpu.sync_copy(idx_hbm.at[wid], idx_v)

        def scatter_wait(buf, sem):
            pltpu.make_async_copy(buf, accum_sh.at[idx_v.at[0]], sem).wait()

        def fetch_wait(jj, buf, sem):
            pltpu.make_async_copy(feat_hbm.at[wid, jj], buf, sem).wait()

        pltpu.async_copy(feat_hbm.at[wid, 0], bufs[0], fsems[0])
        pltpu.async_copy(feat_hbm.at[wid, 1], bufs[1], fsems[1])

        def step(t, carry):
            for r in range(3):
                k = 3 * t + r

                def sub(kk, b_next, b_cur):
                    scatter_wait(bufs[b_next], ssems[b_next])
                    pltpu.async_copy(feat_hbm.at[wid, kk + 2],
                                     bufs[b_next], fsems[b_next])
                    fetch_wait(kk, bufs[b_cur], fsems[b_cur])
                    pltpu.async_copy(bufs[b_cur],
                                     accum_sh.at[idx_v.at[kk]],
                                     ssems[b_cur], add=True)

                if r == 0:
                    @pl.when(t > 0)
                    def _():
                        sub(k, (r + 2) % 3, r)

                    @pl.when(t == 0)
                    def _():
                        pltpu.async_copy(feat_hbm.at[wid, k + 2],
                                         bufs[2], fsems[2])
                        fetch_wait(k, bufs[0], fsems[0])
                        pltpu.async_copy(bufs[0],
                                         accum_sh.at[idx_v.at[k]],
                                         ssems[0], add=True)
                else:
                    sub(k, (r + 2) % 3, r)
            return carry

        n_full = (STEPS - 2) // 3
        lax.fori_loop(0, n_full, step, 0)
        k0 = 3 * n_full
        scatter_wait(bufs[2], ssems[2])
        fetch_wait(k0, bufs[0], fsems[0])
        pltpu.async_copy(bufs[0], accum_sh.at[idx_v.at[k0]], ssems[0],
                         add=True)
        fetch_wait(k0 + 1, bufs[1], fsems[1])
        pltpu.async_copy(bufs[1], accum_sh.at[idx_v.at[k0 + 1]], ssems[1],
                         add=True)
        scatter_wait(bufs[0], ssems[0])
        scatter_wait(bufs[1], ssems[1])
        plsc.subcore_barrier()

        for kk in range(ZITERS):
            q = kk * N_SUB + s

            @pl.when(q < ZCHUNKS)
            def _():
                pltpu.sync_copy(accum_sh.at[pl.ds(q * ZROWS, ZROWS)], rows_a)
                pltpu.sync_copy(rows_a, part_hbm.at[c, q])

    return k(features4, idx3, zeros_stage)


def _combine_body(a_ref, b_ref, o_ref):
    o_ref[...] = a_ref[...] + b_ref[...]


_COMBINE_BLK = 1000


def _combine(p0, p1):
    grid = N_MOLS // _COMBINE_BLK
    spec = pl.BlockSpec((_COMBINE_BLK, D_FEAT), lambda i: (i, 0))
    return pl.pallas_call(
        _combine_body,
        grid=(grid,),
        in_specs=[spec, spec],
        out_specs=spec,
        out_shape=jax.ShapeDtypeStruct((N_MOLS, D_FEAT), jnp.float32),
    )(p0, p1)


def kernel(features, mol_index, n_molecules):
    del n_molecules
    feat4 = features.reshape(NW, STEPS, CHUNK, D_FEAT)
    idx3 = mol_index.astype(jnp.int32).reshape(NW, STEPS, CHUNK)
    zeros_stage = jnp.zeros((ZROWS, D_FEAT), jnp.float32)
    part = _sc_partials(feat4, idx3, zeros_stage)
    part = part.reshape(N_CORES, N_MOLS, D_FEAT)
    return _combine(part[0], part[1])


# speedup vs baseline: 2.5101x; 1.0320x over previous
import functools

import jax
import jax.numpy as jnp
from jax import lax
from jax.experimental import pallas as pl
from jax.experimental.pallas import tpu as pltpu
from jax.experimental.pallas import tpu_sc as plsc

N_ATOMS = 320000
D_FEAT = 128
N_MOLS = 10000

N_CORES = 2
N_SUB = 16
NW = N_CORES * N_SUB
PER_W = N_ATOMS // NW
CHUNK = 80
STEPS = PER_W // CHUNK
NBUF = 4
ZROWS = 80
ZCHUNKS = N_MOLS // ZROWS
ZITERS = -(-ZCHUNKS // N_SUB)


def _sc_partials(features4, idx3, zeros_stage):
    mesh = plsc.VectorSubcoreMesh(core_axis_name="c", subcore_axis_name="s")

    @functools.partial(
        pl.kernel,
        out_type=jax.ShapeDtypeStruct((N_CORES, ZCHUNKS, ZROWS, D_FEAT),
                                      jnp.float32),
        mesh=mesh,
        scratch_types=(
            [pltpu.VMEM((CHUNK, D_FEAT), jnp.float32)] * NBUF
            + [pltpu.VMEM((CHUNK,), jnp.int32)] * NBUF
            + [pltpu.VMEM_SHARED((N_MOLS, D_FEAT), jnp.float32)]
            + [pltpu.SemaphoreType.DMA] * NBUF
            + [pltpu.SemaphoreType.DMA] * NBUF
        ),
    )
    def k(feat_hbm, idx_hbm, zero_hbm, part_hbm, *scratch):
        bufs = scratch[:NBUF]
        idxb = scratch[NBUF:2 * NBUF]
        accum_sh = scratch[2 * NBUF]
        ssems = scratch[2 * NBUF + 1:3 * NBUF + 1]
        fsems = scratch[3 * NBUF + 1:4 * NBUF + 1]
        rows_a = bufs[0]
        c = lax.axis_index("c")
        s = lax.axis_index("s")
        wid = c * N_SUB + s

        pltpu.sync_copy(zero_hbm, rows_a)
        for kk in range(ZITERS):
            q = kk * N_SUB + s

            @pl.when(q < ZCHUNKS)
            def _():
                pltpu.sync_copy(rows_a, accum_sh.at[pl.ds(q * ZROWS, ZROWS)])

        plsc.subcore_barrier()

        def fetch(kk, b):
            pltpu.async_copy(feat_hbm.at[wid, kk], bufs[b], fsems[b])
            pltpu.async_copy(idx_hbm.at[pl.ds(wid * PER_W + kk * CHUNK,
                                              CHUNK)],
                             idxb[b], fsems[b])

        def fetch_wait(kk, b):
            pltpu.make_async_copy(feat_hbm.at[wid, kk], bufs[b],
                                  fsems[b]).wait()
            pltpu.make_async_copy(idx_hbm.at[pl.ds(0, CHUNK)], idxb[b],
                                  fsems[b]).wait()

        def scatter(kk, b):
            pltpu.async_copy(bufs[b], accum_sh.at[idxb[b]], ssems[b],
                             add=True)

        def scatter_wait(b):
            pltpu.make_async_copy(bufs[b], accum_sh.at[idxb[b]],
                                  ssems[b]).wait()

        for b in range(NBUF - 1):
            fetch(b, b)

        def substep(kk, r, do_free):
            b_next = (r + NBUF - 1) % NBUF
            if do_free:
                scatter_wait(b_next)
                fetch(kk + NBUF - 1, b_next)
            fetch_wait(kk, r)
            scatter(kk, r)

        def step(t, carry):
            for r in range(NBUF):
                k = NBUF * t + r
                if r == 0:
                    @pl.when(t > 0)
                    def _():
                        substep(k, 0, True)

                    @pl.when(t == 0)
                    def _():
                        fetch(NBUF - 1, NBUF - 1)
                        fetch_wait(0, 0)
                        scatter(0, 0)
                else:
                    substep(k, r, True)
            return carry

        n_full = (STEPS - NBUF + 1) // NBUF
        lax.fori_loop(0, n_full, step, 0)
        for k in range(NBUF * n_full, STEPS):
            r = k % NBUF
            b_next = (r + NBUF - 1) % NBUF
            scatter_wait(b_next)
            if k + NBUF - 1 < STEPS:
                fetch(k + NBUF - 1, b_next)
            fetch_wait(k, r)
            scatter(k, r)
        scatter_wait((STEPS - 1) % NBUF)
        plsc.subcore_barrier()

        for kk in range(ZITERS):
            q = kk * N_SUB + s

            @pl.when(q < ZCHUNKS)
            def _():
                pltpu.sync_copy(accum_sh.at[pl.ds(q * ZROWS, ZROWS)], rows_a)
                pltpu.sync_copy(rows_a, part_hbm.at[c, q])

    return k(features4, idx3, zeros_stage)


def _combine_body(a_ref, b_ref, o_ref):
    o_ref[...] = a_ref[...] + b_ref[...]


_COMBINE_BLK = 1000


def _combine(p0, p1):
    grid = N_MOLS // _COMBINE_BLK
    spec = pl.BlockSpec((_COMBINE_BLK, D_FEAT), lambda i: (i, 0))
    return pl.pallas_call(
        _combine_body,
        grid=(grid,),
        in_specs=[spec, spec],
        out_specs=spec,
        out_shape=jax.ShapeDtypeStruct((N_MOLS, D_FEAT), jnp.float32),
    )(p0, p1)


def kernel(features, mol_index, n_molecules):
    del n_molecules
    feat4 = features.reshape(NW, STEPS, CHUNK, D_FEAT)
    idx_flat = mol_index.astype(jnp.int32)
    zeros_stage = jnp.zeros((ZROWS, D_FEAT), jnp.float32)
    part = _sc_partials(feat4, idx_flat, zeros_stage)
    part = part.reshape(N_CORES, N_MOLS, D_FEAT)
    return _combine(part[0], part[1])


# speedup vs baseline: 2.5429x; 1.0131x over previous
import functools

import jax
import jax.numpy as jnp
from jax import lax
from jax.experimental import pallas as pl
from jax.experimental.pallas import tpu as pltpu
from jax.experimental.pallas import tpu_sc as plsc

N_ATOMS = 320000
D_FEAT = 128
N_MOLS = 10000

N_CORES = 2
N_SUB = 16
NW = N_CORES * N_SUB
PER_W = N_ATOMS // NW
CHUNK = 80
STEPS = PER_W // CHUNK
NBUF = 4
ZROWS = 80
ZCHUNKS = N_MOLS // ZROWS
ZITERS = -(-ZCHUNKS // N_SUB)


def _sc_partials(features4, idx3, zeros_stage):
    mesh = plsc.VectorSubcoreMesh(core_axis_name="c", subcore_axis_name="s")

    @functools.partial(
        pl.kernel,
        out_type=jax.ShapeDtypeStruct((N_CORES, ZCHUNKS, ZROWS, D_FEAT),
                                      jnp.float32),
        mesh=mesh,
        scratch_types=(
            [pltpu.VMEM((CHUNK, D_FEAT), jnp.float32)] * NBUF
            + [pltpu.VMEM((CHUNK,), jnp.int32)] * NBUF
            + [pltpu.VMEM_SHARED((N_MOLS, D_FEAT), jnp.float32)]
            + [pltpu.SemaphoreType.DMA] * NBUF
            + [pltpu.SemaphoreType.DMA] * NBUF
        ),
    )
    def k(feat_hbm, idx_hbm, zero_hbm, part_hbm, *scratch):
        bufs = scratch[:NBUF]
        idxb = scratch[NBUF:2 * NBUF]
        accum_sh = scratch[2 * NBUF]
        ssems = scratch[2 * NBUF + 1:3 * NBUF + 1]
        fsems = scratch[3 * NBUF + 1:4 * NBUF + 1]
        c = lax.axis_index("c")
        s = lax.axis_index("s")
        wid = c * N_SUB + s


        def fetch(kk, b):
            pltpu.async_copy(feat_hbm.at[wid, kk], bufs[b], fsems[b])
            pltpu.async_copy(idx_hbm.at[pl.ds(wid * PER_W + kk * CHUNK,
                                              CHUNK)],
                             idxb[b], fsems[b])

        def fetch_wait(kk, b):
            pltpu.make_async_copy(feat_hbm.at[wid, kk], bufs[b],
                                  fsems[b]).wait()
            pltpu.make_async_copy(idx_hbm.at[pl.ds(0, CHUNK)], idxb[b],
                                  fsems[b]).wait()

        def scatter(kk, b):
            pltpu.async_copy(bufs[b], accum_sh.at[idxb[b]], ssems[b],
                             add=True)

        def scatter_wait(b):
            pltpu.make_async_copy(bufs[b], accum_sh.at[idxb[b]],
                                  ssems[b]).wait()

        for b in range(NBUF - 1):
            fetch(b, b)

        pltpu.sync_copy(zero_hbm, bufs[NBUF - 1])
        for kk in range(ZITERS):
            q = kk * N_SUB + s

            @pl.when(q < ZCHUNKS)
            def _():
                pltpu.sync_copy(bufs[NBUF - 1],
                                accum_sh.at[pl.ds(q * ZROWS, ZROWS)])

        plsc.subcore_barrier()

        def substep(kk, r, do_free):
            b_next = (r + NBUF - 1) % NBUF
            if do_free:
                scatter_wait(b_next)
                fetch(kk + NBUF - 1, b_next)
            fetch_wait(kk, r)
            scatter(kk, r)

        def step(t, carry):
            for r in range(NBUF):
                k = NBUF * t + r
                if r == 0:
                    @pl.when(t > 0)
                    def _():
                        substep(k, 0, True)

                    @pl.when(t == 0)
                    def _():
                        fetch(NBUF - 1, NBUF - 1)
                        fetch_wait(0, 0)
                        scatter(0, 0)
                else:
                    substep(k, r, True)
            return carry

        n_full = (STEPS - NBUF + 1) // NBUF
        lax.fori_loop(0, n_full, step, 0)
        for k in range(NBUF * n_full, STEPS):
            r = k % NBUF
            b_next = (r + NBUF - 1) % NBUF
            scatter_wait(b_next)
            if k + NBUF - 1 < STEPS:
                fetch(k + NBUF - 1, b_next)
            fetch_wait(k, r)
            scatter(k, r)
        scatter_wait((STEPS - 1) % NBUF)
        plsc.subcore_barrier()

        def flush_wait(b):
            pltpu.make_async_copy(bufs[b], part_hbm.at[c, 0],
                                  fsems[b]).wait()

        for kk in range(ZITERS):
            b = kk % 2
            if kk >= 2:
                flush_wait(b)
            q = kk * N_SUB + s

            def issue(qq=q, bb=b):
                pltpu.sync_copy(accum_sh.at[pl.ds(qq * ZROWS, ZROWS)],
                                bufs[bb])
                pltpu.async_copy(bufs[bb], part_hbm.at[c, qq], fsems[bb])

            if (kk + 1) * N_SUB <= ZCHUNKS:
                issue()
            else:
                @pl.when(q < ZCHUNKS)
                def _():
                    issue()

        flush_wait((ZITERS - 2) % 2)

        @pl.when(s < ZCHUNKS - (ZITERS - 1) * N_SUB)
        def _():
            flush_wait((ZITERS - 1) % 2)

    return k(features4, idx3, zeros_stage)


def _combine_body(a_ref, b_ref, o_ref):
    o_ref[...] = a_ref[...] + b_ref[...]


_COMBINE_BLK = 1000


def _combine(p0, p1):
    grid = N_MOLS // _COMBINE_BLK
    spec = pl.BlockSpec((_COMBINE_BLK, D_FEAT), lambda i: (i, 0))
    return pl.pallas_call(
        _combine_body,
        grid=(grid,),
        in_specs=[spec, spec],
        out_specs=spec,
        out_shape=jax.ShapeDtypeStruct((N_MOLS, D_FEAT), jnp.float32),
    )(p0, p1)


def kernel(features, mol_index, n_molecules):
    del n_molecules
    feat4 = features.reshape(NW, STEPS, CHUNK, D_FEAT)
    idx_flat = mol_index.astype(jnp.int32)
    zeros_stage = jnp.zeros((ZROWS, D_FEAT), jnp.float32)
    part = _sc_partials(feat4, idx_flat, zeros_stage)
    part = part.reshape(N_CORES, N_MOLS, D_FEAT)
    return _combine(part[0], part[1])


# speedup vs baseline: 2.5738x; 1.0121x over previous
import functools

import jax
import jax.numpy as jnp
from jax import lax
from jax.experimental import pallas as pl
from jax.experimental.pallas import tpu as pltpu
from jax.experimental.pallas import tpu_sc as plsc

N_ATOMS = 320000
D_FEAT = 128
N_MOLS = 10000

N_CORES = 2
N_SUB = 16
NW = N_CORES * N_SUB
PER_W = N_ATOMS // NW
CHUNK = 80
STEPS = PER_W // CHUNK
NBUF = 4
ZROWS = 80
ZCHUNKS = N_MOLS // ZROWS
ZITERS = -(-ZCHUNKS // N_SUB)


def _sc_partials(features4, idx3, zeros_stage):
    mesh = plsc.VectorSubcoreMesh(core_axis_name="c", subcore_axis_name="s")

    @functools.partial(
        pl.kernel,
        out_type=jax.ShapeDtypeStruct((N_CORES, ZCHUNKS, ZROWS, D_FEAT),
                                      jnp.float32),
        mesh=mesh,
        scratch_types=(
            [pltpu.VMEM((CHUNK, D_FEAT), jnp.float32)] * NBUF
            + [pltpu.VMEM((CHUNK,), jnp.int32)] * NBUF
            + [pltpu.VMEM_SHARED((N_MOLS, D_FEAT), jnp.float32)]
            + [pltpu.SemaphoreType.DMA] * NBUF
            + [pltpu.SemaphoreType.DMA] * NBUF
        ),
    )
    def k(feat_hbm, idx_hbm, zero_hbm, part_hbm, *scratch):
        bufs = scratch[:NBUF]
        idxb = scratch[NBUF:2 * NBUF]
        accum_sh = scratch[2 * NBUF]
        ssems = scratch[2 * NBUF + 1:3 * NBUF + 1]
        fsems = scratch[3 * NBUF + 1:4 * NBUF + 1]
        c = lax.axis_index("c")
        s = lax.axis_index("s")
        wid = c * N_SUB + s


        def fetch(kk, b):
            pltpu.async_copy(feat_hbm.at[wid, kk], bufs[b], fsems[b])
            pltpu.async_copy(idx_hbm.at[pl.ds(wid * PER_W + kk * CHUNK,
                                              CHUNK)],
                             idxb[b], fsems[b])

        def fetch_wait(kk, b):
            pltpu.make_async_copy(feat_hbm.at[wid, kk], bufs[b],
                                  fsems[b]).wait()
            pltpu.make_async_copy(idx_hbm.at[pl.ds(0, CHUNK)], idxb[b],
                                  fsems[b]).wait()

        def scatter(kk, b):
            pltpu.async_copy(bufs[b], accum_sh.at[idxb[b]], ssems[b],
                             add=True)

        def scatter_wait(b):
            pltpu.make_async_copy(bufs[b], accum_sh.at[idxb[b]],
                                  ssems[b]).wait()

        for b in range(NBUF - 1):
            fetch(b, b)

        pltpu.sync_copy(zero_hbm, bufs[NBUF - 1])
        for kk in range(ZITERS):
            q = kk * N_SUB + s

            @pl.when(q < ZCHUNKS)
            def _():
                pltpu.sync_copy(bufs[NBUF - 1],
                                accum_sh.at[pl.ds(q * ZROWS, ZROWS)])

        plsc.subcore_barrier()

        def substep(kk, r, do_free):
            b_next = (r + NBUF - 1) % NBUF
            if do_free:
                scatter_wait(b_next)
                fetch(kk + NBUF - 1, b_next)
            fetch_wait(kk, r)
            scatter(kk, r)

        def step(t, carry):
            for r in range(NBUF):
                k = NBUF * t + r
                if r == 0:
                    @pl.when(t > 0)
                    def _():
                        substep(k, 0, True)

                    @pl.when(t == 0)
                    def _():
                        fetch(NBUF - 1, NBUF - 1)
                        fetch_wait(0, 0)
                        scatter(0, 0)
                else:
                    substep(k, r, True)
            return carry

        n_full = (STEPS - NBUF + 1) // NBUF
        lax.fori_loop(0, n_full, step, 0)
        for k in range(NBUF * n_full, STEPS):
            r = k % NBUF
            b_next = (r + NBUF - 1) % NBUF
            scatter_wait(b_next)
            if k + NBUF - 1 < STEPS:
                fetch(k + NBUF - 1, b_next)
            fetch_wait(k, r)
            scatter(k, r)
        scatter_wait((STEPS - 1) % NBUF)
        plsc.subcore_barrier()

        def flush_wait(b):
            pltpu.make_async_copy(bufs[b], part_hbm.at[c, 0],
                                  fsems[b]).wait()

        for kk in range(ZITERS):
            b = kk % 2
            if kk >= 2:
                flush_wait(b)
            q = kk * N_SUB + s

            def issue(qq=q, bb=b):
                pltpu.sync_copy(accum_sh.at[pl.ds(qq * ZROWS, ZROWS)],
                                bufs[bb])
                pltpu.async_copy(bufs[bb], part_hbm.at[c, qq], fsems[bb])

            if (kk + 1) * N_SUB <= ZCHUNKS:
                issue()
            else:
                @pl.when(q < ZCHUNKS)
                def _():
                    issue()

        flush_wait((ZITERS - 2) % 2)

        @pl.when(s < ZCHUNKS - (ZITERS - 1) * N_SUB)
        def _():
            flush_wait((ZITERS - 1) % 2)

    return k(features4, idx3, zeros_stage)


def _combine_body(a_ref, b_ref, o_ref):
    o_ref[...] = a_ref[...] + b_ref[...]


_COMBINE_BLK = 2000


def _combine(p0, p1):
    grid = N_MOLS // _COMBINE_BLK
    spec = pl.BlockSpec((_COMBINE_BLK, D_FEAT), lambda i: (i, 0))
    return pl.pallas_call(
        _combine_body,
        grid=(grid,),
        in_specs=[spec, spec],
        out_specs=spec,
        out_shape=jax.ShapeDtypeStruct((N_MOLS, D_FEAT), jnp.float32),
    )(p0, p1)


def kernel(features, mol_index, n_molecules):
    del n_molecules
    feat4 = features.reshape(NW, STEPS, CHUNK, D_FEAT)
    idx_flat = mol_index.astype(jnp.int32)
    zeros_stage = jnp.zeros((ZROWS, D_FEAT), jnp.float32)
    part = _sc_partials(feat4, idx_flat, zeros_stage)
    part = part.reshape(N_CORES, N_MOLS, D_FEAT)
    return _combine(part[0], part[1])
